# baseline (speedup 1.0000x reference)
import jax
import jax.numpy as jnp
from jax import lax
from jax.experimental import pallas as pl
from jax.experimental.pallas import tpu as pltpu
from jax.experimental.pallas import tpu_sc as plsc

B, K, D = 1024, 50, 32
KP = 64
NC, NS = 2, 16
NW = NC * NS
BPW = B // NW
HALF = D // 2


def _dis_loss_body(emb_hbm, table_hbm, attr_hbm, idx_hbm, out_hbm,
                   embb_v, attr_v, idx_v, rows_v, o_v, sem):
    wid = lax.axis_index("s") * NC + lax.axis_index("c")
    base = wid * BPW
    pltpu.sync_copy(emb_hbm.at[pl.ds(base, BPW)], embb_v)
    pltpu.sync_copy(attr_hbm.at[pl.ds(base, BPW)], attr_v)
    pltpu.sync_copy(idx_hbm.at[pl.ds(base, BPW)], idx_v)

    def b_loop(b, carry):
        acc_lo, acc_hi = carry
        pltpu.async_copy(table_hbm.at[idx_v.at[b]], rows_v, sem).wait()
        x_lo = embb_v[b, 0:HALF]
        x_hi = embb_v[b, HALF:D]
        for g in range(KP // HALF):
            av = attr_v[b, g * HALF:(g + 1) * HALF]
            for kk in range(HALF):
                a = av[kk]
                k = g * HALF + kk
                d_lo = rows_v[k, 0:HALF] - x_lo
                d_hi = rows_v[k, HALF:D] - x_hi
                acc_lo = acc_lo + a * (d_lo * d_lo)
                acc_hi = acc_hi + a * (d_hi * d_hi)
        return (acc_lo, acc_hi)

    z = jnp.zeros((HALF,), jnp.float32)
    acc_lo, acc_hi = lax.fori_loop(0, BPW, b_loop, (z, z))
    o_v[...] = acc_lo + acc_hi
    pltpu.sync_copy(o_v, out_hbm.at[wid])


def kernel(emb_batch, embedding, attr_sim, indices, beta):
    del beta
    attr_p = jnp.pad(attr_sim, ((0, 0), (0, KP - K)))
    idx_p = jnp.pad(indices, ((0, 0), (0, KP - K)))
    mesh = plsc.VectorSubcoreMesh(core_axis_name="c", subcore_axis_name="s")
    out = pl.kernel(
        _dis_loss_body,
        out_type=jax.ShapeDtypeStruct((NW, HALF), jnp.float32),
        mesh=mesh,
        compiler_params=pltpu.CompilerParams(use_tc_tiling_on_sc=False),
        scratch_types=[
            pltpu.VMEM((BPW, D), jnp.float32),
            pltpu.VMEM((BPW, KP), jnp.float32),
            pltpu.VMEM((BPW, KP), jnp.int32),
            pltpu.VMEM((KP, D), jnp.float32),
            pltpu.VMEM((HALF,), jnp.float32),
            pltpu.SemaphoreType.DMA,
        ],
    )(emb_batch, embedding, attr_p, idx_p)
    return jnp.sum(out) / jnp.float32(B)
